# baseline (speedup 1.0000x reference)
import functools

import jax
import jax.numpy as jnp
from jax import lax
from jax.experimental import pallas as pl
from jax.experimental.pallas import tpu as pltpu
from jax.experimental.pallas import tpu_sc as plsc

_R, _D = 1_000_000, 64
_N = 16384
_FLAT = _N * _D

_NC, _NS, _L = 2, 16, 16
_NW = _NC * _NS

_CH = 128
_ROWS = _FLAT // _CH

_SEGS = (48, 16)
_J0 = (0, 48)

_K = 384
_BW = _K * _CH
_NB = -(-7813 // _K)


def _make_relayout_body(jg):
    def _relayout_body(in_ref, out_ref):
        x = in_ref[...].reshape(jg, _K, _CH)
        out_ref[...] = jnp.swapaxes(x, 0, 1).reshape(jg * _BW)

    return _relayout_body


@functools.cache
def _relayout_kernel(g):
    jg, j0 = _SEGS[g], _J0[g]
    return pl.pallas_call(
        _make_relayout_body(jg),
        grid=(_NB,),
        in_specs=[pl.BlockSpec((jg, _BW), lambda c: (j0 // jg, c))],
        out_specs=pl.BlockSpec((jg * _BW,), lambda c: (c,)),
        out_shape=jax.ShapeDtypeStruct((_NB * jg * _BW,), jnp.float32),
    )


def _make_gather_body(g):
    jg, j0 = _SEGS[g], _J0[g]
    srows = jg * _CH
    snr = srows // _NW

    def _gather_body(t_hbm, c_hbm, out_hbm, ibuf, gbuf, sem):
        wid = lax.axis_index("s") * _NC + lax.axis_index("c")
        grow0 = j0 * _CH + wid * snr
        row0 = wid * snr
        pltpu.sync_copy(c_hbm.at[pl.ds(grow0, snr), :], ibuf)

        def fire(r, carry):
            jconst = (((grow0 + r) >> 7) - j0) * _CH
            for m in range(_CH // _L):
                sl = pl.ds(m * _L, _L)
                v = ibuf[r, sl]
                ibuf[r, sl] = (v & ~jnp.int32(127)) * jg + ((v & 127) + jconst)
            pltpu.async_copy(t_hbm.at[ibuf.at[r]], gbuf.at[r], sem)
            return carry

        lax.fori_loop(0, snr, fire, 0)
        pltpu.make_async_copy(out_hbm.at[pl.ds(row0, snr), :], gbuf, sem).wait()
        pltpu.sync_copy(gbuf, out_hbm.at[pl.ds(row0, snr), :])

    return _gather_body


@functools.cache
def _gather_kernel(g):
    jg = _SEGS[g]
    srows = jg * _CH
    snr = srows // _NW
    mesh = plsc.VectorSubcoreMesh(
        core_axis_name="c", subcore_axis_name="s", num_cores=_NC, num_subcores=_NS
    )
    return pl.kernel(
        _make_gather_body(g),
        mesh=mesh,
        out_type=jax.ShapeDtypeStruct((srows, _CH), jnp.float32),
        scratch_types=[
            pltpu.VMEM((snr, _CH), jnp.int32),
            pltpu.VMEM((snr, _CH), jnp.float32),
            pltpu.SemaphoreType.DMA,
        ],
    )


def kernel(t, d, c):
    idx = c + jnp.asarray(d, dtype=c.dtype)
    cflat = idx.T.reshape(_ROWS, _CH)
    tt = t.T
    outs = [
        _gather_kernel(g)(_relayout_kernel(g)(tt), cflat)
        for g in range(len(_SEGS))
    ]
    out = jnp.concatenate(outs, axis=0)
    return out.reshape(_D, _N).T


# speedup vs baseline: 1.0151x; 1.0151x over previous
import functools

import jax
import jax.numpy as jnp
from jax import lax
from jax.experimental import pallas as pl
from jax.experimental.pallas import tpu as pltpu
from jax.experimental.pallas import tpu_sc as plsc

_R, _D = 1_000_000, 64
_N = 16384
_FLAT = _N * _D

_NC, _NS, _L = 2, 16, 16
_NW = _NC * _NS

_CH = 128
_ROWS = _FLAT // _CH

_G = 2
_JG = _D // _G
_SROWS = _ROWS // _G
_SNR = _SROWS // _NW

_K = 384
_BW = _K * _CH
_NB = -(-7813 // _K)
_OB = _JG * _BW
_TSEG = _NB * _OB


def _relayout_body(in_ref, out_ref):
    x = in_ref[...].reshape(_JG, _K, _CH)
    out_ref[...] = jnp.swapaxes(x, 0, 1).reshape(_OB)


@functools.cache
def _relayout_kernel(g):
    return pl.pallas_call(
        _relayout_body,
        grid=(_NB,),
        in_specs=[pl.BlockSpec((_JG, _BW), lambda c: (g, c))],
        out_specs=pl.BlockSpec((_OB,), lambda c: (c,)),
        out_shape=jax.ShapeDtypeStruct((_TSEG,), jnp.float32),
    )


def _make_gather_body(g):
    def _gather_body(t_hbm, c_hbm, out_hbm, ibuf, gbuf, sem):
        wid = lax.axis_index("s") * _NC + lax.axis_index("c")
        grow0 = g * _SROWS + wid * _SNR
        row0 = wid * _SNR
        pltpu.sync_copy(c_hbm.at[pl.ds(grow0, _SNR), :], ibuf)

        def fire(r, carry):
            jconst = (((grow0 + r) >> 7) & (_JG - 1)) * _CH
            for m in range(_CH // _L):
                sl = pl.ds(m * _L, _L)
                v = ibuf[r, sl]
                ibuf[r, sl] = ((v & ~jnp.int32(127)) << 5) + ((v & 127) + jconst)
            pltpu.async_copy(t_hbm.at[ibuf.at[r]], gbuf.at[r], sem)
            return carry

        lax.fori_loop(0, _SNR, fire, 0)
        pltpu.make_async_copy(out_hbm.at[pl.ds(row0, _SNR), :], gbuf, sem).wait()
        pltpu.sync_copy(gbuf, out_hbm.at[pl.ds(row0, _SNR), :])

    return _gather_body


@functools.cache
def _gather_kernel(g):
    mesh = plsc.VectorSubcoreMesh(
        core_axis_name="c", subcore_axis_name="s", num_cores=_NC, num_subcores=_NS
    )
    return pl.kernel(
        _make_gather_body(g),
        mesh=mesh,
        out_type=jax.ShapeDtypeStruct((_SROWS, _CH), jnp.float32),
        scratch_types=[
            pltpu.VMEM((_SNR, _CH), jnp.int32),
            pltpu.VMEM((_SNR, _CH), jnp.float32),
            pltpu.SemaphoreType.DMA,
        ],
    )


def kernel(t, d, c):
    idx = c + jnp.asarray(d, dtype=c.dtype)
    cflat = idx.T.reshape(_ROWS, _CH)
    tt = t.T
    outs = [_gather_kernel(g)(_relayout_kernel(g)(tt), cflat) for g in range(_G)]
    out = jnp.concatenate(outs, axis=0)
    return out.reshape(_D, _N).T
